# Initial kernel scaffold; baseline (speedup 1.0000x reference)
import jax
import jax.numpy as jnp
from jax.experimental import pallas as pl


def kernel(i_goods_id, i_shop_id, i_cate_id, visited_goods_ids, visited_shop_ids, visited_cate_ids, emb_table, W1, b1, g1, be1, a1, W2, b2, g2, be2, a2, W3, b3):
    raise NotImplementedError("write your pallas kernel here")


# speedup vs baseline: 1.2811x; 1.2811x over previous
import functools

import jax
import jax.numpy as jnp
from jax import lax
from jax.experimental import pallas as pl
from jax.experimental.pallas import tpu as pltpu
from jax.experimental.pallas import tpu_sc as plsc

B, L, E, V = 4096, 200, 16, 1000000
H1, H2, OUT = 200, 80, 2
D_IN = 6 * E

NW = 32
S_ROWS = B * 3 * L
I_ROWS = 3 * B
S_PER_W = S_ROWS // NW
I_PER_W = I_ROWS // NW
CHUNK = 1024
N_CHUNKS = S_PER_W // CHUNK


def _sc_gather(table, sidx, iidx):
  mesh = plsc.VectorSubcoreMesh(core_axis_name="c", subcore_axis_name="s")

  @functools.partial(
      pl.kernel,
      out_type=[
          jax.ShapeDtypeStruct((S_ROWS, E), jnp.float32),
          jax.ShapeDtypeStruct((I_ROWS, E), jnp.float32),
      ],
      mesh=mesh,
      compiler_params=pltpu.CompilerParams(use_tc_tiling_on_sc=False),
      scratch_types=[
          pltpu.VMEM((CHUNK,), jnp.int32),
          pltpu.VMEM((CHUNK, E), jnp.float32),
          pltpu.VMEM((I_PER_W,), jnp.int32),
          pltpu.VMEM((I_PER_W, E), jnp.float32),
          pltpu.SemaphoreType.DMA,
      ],
  )
  def k(table_hbm, sidx_hbm, iidx_hbm, srows_hbm, irows_hbm,
        idx_v, rows_v, iidx_v, irows_v, sem):
    wid = lax.axis_index("s") * 2 + lax.axis_index("c")

    ibase = wid * I_PER_W
    pltpu.sync_copy(iidx_hbm.at[pl.ds(ibase, I_PER_W)], iidx_v)
    pltpu.async_copy(table_hbm.at[iidx_v], irows_v, sem).wait()
    pltpu.sync_copy(irows_v, irows_hbm.at[pl.ds(ibase, I_PER_W)])

    sbase = wid * S_PER_W

    def body(i, carry):
      off = sbase + i * CHUNK
      pltpu.sync_copy(sidx_hbm.at[pl.ds(off, CHUNK)], idx_v)
      pltpu.async_copy(table_hbm.at[idx_v], rows_v, sem).wait()
      pltpu.sync_copy(rows_v, srows_hbm.at[pl.ds(off, CHUNK)])
      return carry

    lax.fori_loop(0, N_CHUNKS, body, 0)

  return k(table, sidx, iidx)


BB = 128


def _attn(x_item, x_series, vg):
  def body(xi_ref, xs_ref, vg_ref, x_ref):
    xi = xi_ref[...]
    xs = xs_ref[...]
    m = (vg_ref[...] != 0).astype(jnp.float32)
    scores = jnp.sum(xs * xi[:, None, :], axis=-1)
    ms = scores * m
    pooled = jnp.sum(ms[:, :, None] * xs, axis=1)
    x_ref[...] = jnp.concatenate([xi, pooled], axis=-1)

  return pl.pallas_call(
      body,
      grid=(B // BB,),
      in_specs=[
          pl.BlockSpec((BB, 3 * E), lambda i: (i, 0)),
          pl.BlockSpec((BB, L, 3 * E), lambda i: (i, 0, 0)),
          pl.BlockSpec((BB, L), lambda i: (i, 0)),
      ],
      out_specs=pl.BlockSpec((BB, D_IN), lambda i: (i, 0)),
      out_shape=jax.ShapeDtypeStruct((B, D_IN), jnp.float32),
  )(x_item, x_series, vg)


def _mlp(X, W1, b1, g1, be1, a1, W2, b2, g2, be2, a2, W3, b3):
  eps = 1e-3

  def ln(h, g, be):
    mu = jnp.mean(h, axis=-1, keepdims=True)
    var = jnp.mean((h - mu) * (h - mu), axis=-1, keepdims=True)
    return g * (h - mu) * lax.rsqrt(var + eps) + be

  def dice(h, a):
    mu = jnp.mean(h, axis=0, keepdims=True)
    var = jnp.mean((h - mu) * (h - mu), axis=0, keepdims=True)
    xn = (h - mu) * lax.rsqrt(var + eps)
    p = jax.nn.sigmoid(xn)
    return a * (1.0 - p) * h + p * h

  def body(x_ref, w1, b1_, g1_, be1_, a1_, w2, b2_, g2_, be2_, a2_, w3, b3_,
           o_ref):
    x = x_ref[...]
    h = jnp.dot(x, w1[...], preferred_element_type=jnp.float32) + b1_[...]
    h = ln(h, g1_[...], be1_[...])
    h = dice(h, a1_[...])
    h = jnp.dot(h, w2[...], preferred_element_type=jnp.float32) + b2_[...]
    h = ln(h, g2_[...], be2_[...])
    h = dice(h, a2_[...])
    logits = jnp.dot(h, w3[...], preferred_element_type=jnp.float32) + b3_[...]
    o_ref[...] = jax.nn.softmax(logits, axis=-1)

  return pl.pallas_call(
      body,
      out_shape=jax.ShapeDtypeStruct((B, OUT), jnp.float32),
  )(X, W1, b1.reshape(1, H1), g1.reshape(1, H1), be1.reshape(1, H1),
    a1.reshape(1, H1), W2, b2.reshape(1, H2), g2.reshape(1, H2),
    be2.reshape(1, H2), a2.reshape(1, H2), W3, b3.reshape(1, OUT))


def kernel(i_goods_id, i_shop_id, i_cate_id, visited_goods_ids,
           visited_shop_ids, visited_cate_ids, emb_table, W1, b1, g1, be1, a1,
           W2, b2, g2, be2, a2, W3, b3):
  sidx = jnp.stack(
      [visited_goods_ids, visited_shop_ids, visited_cate_ids],
      axis=2).reshape(S_ROWS)
  iidx = jnp.stack([i_goods_id, i_shop_id, i_cate_id], axis=1).reshape(I_ROWS)
  srows, irows = _sc_gather(emb_table, sidx, iidx)
  X_series = srows.reshape(B, L, 3 * E)
  X_item = irows.reshape(B, 3 * E)
  valid_mask = visited_goods_ids != 0
  X = _attn(X_item, X_series, visited_goods_ids)
  output = _mlp(X, W1, b1, g1, be1, a1, W2, b2, g2, be2, a2, W3, b3)
  return output, X_series, valid_mask
